# baseline (speedup 1.0000x reference)
import jax
import jax.numpy as jnp
from jax import lax
from jax.experimental import pallas as pl
from jax.experimental.pallas import tpu as pltpu

_N = 262144
_EPS = 1e-7
_ROWS = _N // 128
_DROWS = 4 * _ROWS
_BLK = 256
_DBLK = 4 * _BLK
_STEPS = _ROWS // _BLK
_C = 32
_NCH = _BLK // _C


def _start_copies(ts_hbm, os_hbm, td_hbm, od_hbm, bufs, sems, slot, step):
    ts_b, os_b, td_b, od_b = bufs
    pltpu.make_async_copy(
        ts_hbm.at[pl.ds(step * _BLK, _BLK), :], ts_b.at[slot], sems.at[slot, 0]
    ).start()
    pltpu.make_async_copy(
        os_hbm.at[pl.ds(step * _BLK, _BLK), :], os_b.at[slot], sems.at[slot, 1]
    ).start()
    pltpu.make_async_copy(
        td_hbm.at[pl.ds(step * _DBLK, _DBLK), :], td_b.at[slot], sems.at[slot, 2]
    ).start()
    pltpu.make_async_copy(
        od_hbm.at[pl.ds(step * _DBLK, _DBLK), :], od_b.at[slot], sems.at[slot, 3]
    ).start()


def _wait_copies(ts_hbm, os_hbm, td_hbm, od_hbm, bufs, sems, slot, step):
    ts_b, os_b, td_b, od_b = bufs
    pltpu.make_async_copy(
        ts_hbm.at[pl.ds(step * _BLK, _BLK), :], ts_b.at[slot], sems.at[slot, 0]
    ).wait()
    pltpu.make_async_copy(
        os_hbm.at[pl.ds(step * _BLK, _BLK), :], os_b.at[slot], sems.at[slot, 1]
    ).wait()
    pltpu.make_async_copy(
        td_hbm.at[pl.ds(step * _DBLK, _DBLK), :], td_b.at[slot], sems.at[slot, 2]
    ).wait()
    pltpu.make_async_copy(
        od_hbm.at[pl.ds(step * _DBLK, _DBLK), :], od_b.at[slot], sems.at[slot, 3]
    ).wait()


def _rpn_loss_kernel(ts_hbm, os_hbm, td_hbm, od_hbm, out_ref,
                     ts_b, os_b, td_b, od_b, sems,
                     bce_ref, val_ref, reg_ref, pos_ref):
    i = pl.program_id(0)
    bufs = (ts_b, os_b, td_b, od_b)
    slot = lax.rem(i, 2)

    @pl.when(i == 0)
    def _prologue():
        _start_copies(ts_hbm, os_hbm, td_hbm, od_hbm, bufs, sems, 0, 0)

    @pl.when(i + 1 < _STEPS)
    def _prefetch():
        _start_copies(ts_hbm, os_hbm, td_hbm, od_hbm, bufs, sems,
                      lax.rem(i + 1, 2), i + 1)

    _wait_copies(ts_hbm, os_hbm, td_hbm, od_hbm, bufs, sems, slot, i)

    ts_ref = ts_b.at[slot]
    os_ref = os_b.at[slot]
    td_ref = td_b.at[slot]
    od_ref = od_b.at[slot]

    bce_acc = jnp.zeros((_C, 128), jnp.float32)
    val_acc = jnp.zeros((_C, 128), jnp.float32)
    reg_acc = jnp.zeros((_C, 128), jnp.float32)
    pos_acc = jnp.zeros((_C, 128), jnp.float32)

    for k in range(_NCH):
        ts = ts_ref[k * _C:(k + 1) * _C, :]
        osc = os_ref[k * _C:(k + 1) * _C, :]
        valid = (ts != -1.0).astype(jnp.float32)
        pos = ts > 0.0
        p_star = pos.astype(jnp.float32)
        o = jnp.clip(osc, _EPS, 1.0 - _EPS)
        bce = -jnp.log(jnp.where(pos, o, 1.0 - o))
        bce_acc += bce * valid
        val_acc += valid
        pos_acc += p_star

        for j in range(4):
            r0 = 4 * k * _C + j * _C
            q0 = k * _C + j * (_C // 4)
            mask = jnp.broadcast_to(
                (ts_ref[q0:q0 + _C // 4, :] > 0.0).astype(jnp.float32)[:, None, :],
                (_C // 4, 4, 128)).reshape(_C, 128)
            d = jnp.abs(od_ref[r0:r0 + _C, :] - td_ref[r0:r0 + _C, :])
            m = jnp.minimum(d, 1.0)
            reg_acc += (m * (d - 0.5 * m)) * mask

    @pl.when(i == 0)
    def _init():
        bce_ref[...] = bce_acc
        val_ref[...] = val_acc
        reg_ref[...] = reg_acc
        pos_ref[...] = pos_acc

    @pl.when(i > 0)
    def _accum():
        bce_ref[...] += bce_acc
        val_ref[...] += val_acc
        reg_ref[...] += reg_acc
        pos_ref[...] += pos_acc

    @pl.when(i == _STEPS - 1)
    def _finalize():
        cls_loss = jnp.sum(bce_ref[...]) / jnp.maximum(jnp.sum(val_ref[...]), 1.0)
        reg_loss = 10.0 * jnp.sum(reg_ref[...]) / jnp.maximum(_EPS, jnp.sum(pos_ref[...]))
        out_ref[0, 0] = cls_loss + reg_loss


def kernel(target_deltas, target_scores, output_deltas, output_scores):
    ts = target_scores.reshape(_ROWS, 128)
    osc = output_scores.reshape(_ROWS, 128)
    td = jnp.transpose(target_deltas.reshape(_ROWS, 128, 4), (0, 2, 1)).reshape(_DROWS, 128)
    od = jnp.transpose(output_deltas.reshape(_ROWS, 128, 4), (0, 2, 1)).reshape(_DROWS, 128)

    out = pl.pallas_call(
        _rpn_loss_kernel,
        grid=(_STEPS,),
        in_specs=[
            pl.BlockSpec(memory_space=pl.ANY),
            pl.BlockSpec(memory_space=pl.ANY),
            pl.BlockSpec(memory_space=pl.ANY),
            pl.BlockSpec(memory_space=pl.ANY),
        ],
        out_specs=pl.BlockSpec((1, 1), lambda i: (0, 0), memory_space=pltpu.SMEM),
        out_shape=jax.ShapeDtypeStruct((1, 1), jnp.float32),
        scratch_shapes=[
            pltpu.VMEM((2, _BLK, 128), jnp.float32),
            pltpu.VMEM((2, _BLK, 128), jnp.float32),
            pltpu.VMEM((2, _DBLK, 128), jnp.float32),
            pltpu.VMEM((2, _DBLK, 128), jnp.float32),
            pltpu.SemaphoreType.DMA((2, 4)),
            pltpu.VMEM((_C, 128), jnp.float32),
            pltpu.VMEM((_C, 128), jnp.float32),
            pltpu.VMEM((_C, 128), jnp.float32),
            pltpu.VMEM((_C, 128), jnp.float32),
        ],
        compiler_params=pltpu.CompilerParams(
            dimension_semantics=("arbitrary",),
        ),
    )(ts, osc, td, od)
    return out[0, 0]


# speedup vs baseline: 1.3907x; 1.3907x over previous
import jax
import jax.numpy as jnp
from jax.experimental import pallas as pl
from jax.experimental.pallas import tpu as pltpu

_N = 262144
_EPS = 1e-7
_ROWS = _N // 128
_DROWS = 4 * _ROWS
_BLK = 1024
_DBLK = 4 * _BLK
_STEPS = _ROWS // _BLK
_C = 32
_NCH = _BLK // _C


def _rpn_loss_kernel(ts_ref, os_ref, td_ref, od_ref, out_ref,
                     bce_ref, val_ref, reg_ref, pos_ref):
    i = pl.program_id(0)

    bce_acc = jnp.zeros((_C, 128), jnp.float32)
    val_acc = jnp.zeros((_C, 128), jnp.float32)
    reg_acc = jnp.zeros((_C, 128), jnp.float32)
    pos_acc = jnp.zeros((_C, 128), jnp.float32)

    for k in range(_NCH):
        ts = ts_ref[k * _C:(k + 1) * _C, :]
        osc = os_ref[k * _C:(k + 1) * _C, :]
        valid = (ts != -1.0).astype(jnp.float32)
        pos = ts > 0.0
        p_star = pos.astype(jnp.float32)
        o = jnp.clip(osc, _EPS, 1.0 - _EPS)
        bce = -jnp.log(jnp.where(pos, o, 1.0 - o))
        bce_acc += bce * valid
        val_acc += valid
        pos_acc += p_star

        for j in range(4):
            r0 = 4 * k * _C + j * _C
            q0 = k * _C + j * (_C // 4)
            mask = jnp.broadcast_to(
                (ts_ref[q0:q0 + _C // 4, :] > 0.0).astype(jnp.float32)[:, None, :],
                (_C // 4, 4, 128)).reshape(_C, 128)
            d = jnp.abs(od_ref[r0:r0 + _C, :] - td_ref[r0:r0 + _C, :])
            m = jnp.minimum(d, 1.0)
            reg_acc += (m * (d - 0.5 * m)) * mask

    @pl.when(i == 0)
    def _init():
        bce_ref[...] = bce_acc
        val_ref[...] = val_acc
        reg_ref[...] = reg_acc
        pos_ref[...] = pos_acc

    @pl.when(i > 0)
    def _accum():
        bce_ref[...] += bce_acc
        val_ref[...] += val_acc
        reg_ref[...] += reg_acc
        pos_ref[...] += pos_acc

    @pl.when(i == _STEPS - 1)
    def _finalize():
        cls_loss = jnp.sum(bce_ref[...]) / jnp.maximum(jnp.sum(val_ref[...]), 1.0)
        reg_loss = 10.0 * jnp.sum(reg_ref[...]) / jnp.maximum(_EPS, jnp.sum(pos_ref[...]))
        out_ref[0, 0] = cls_loss + reg_loss


def kernel(target_deltas, target_scores, output_deltas, output_scores):
    ts = target_scores.reshape(_ROWS, 128)
    osc = output_scores.reshape(_ROWS, 128)
    td = jnp.transpose(target_deltas.reshape(_ROWS, 128, 4), (0, 2, 1)).reshape(_DROWS, 128)
    od = jnp.transpose(output_deltas.reshape(_ROWS, 128, 4), (0, 2, 1)).reshape(_DROWS, 128)

    out = pl.pallas_call(
        _rpn_loss_kernel,
        grid=(_STEPS,),
        in_specs=[
            pl.BlockSpec((_BLK, 128), lambda i: (i, 0)),
            pl.BlockSpec((_BLK, 128), lambda i: (i, 0)),
            pl.BlockSpec((_DBLK, 128), lambda i: (i, 0)),
            pl.BlockSpec((_DBLK, 128), lambda i: (i, 0)),
        ],
        out_specs=pl.BlockSpec((1, 1), lambda i: (0, 0), memory_space=pltpu.SMEM),
        out_shape=jax.ShapeDtypeStruct((1, 1), jnp.float32),
        scratch_shapes=[
            pltpu.VMEM((_C, 128), jnp.float32),
            pltpu.VMEM((_C, 128), jnp.float32),
            pltpu.VMEM((_C, 128), jnp.float32),
            pltpu.VMEM((_C, 128), jnp.float32),
        ],
        compiler_params=pltpu.CompilerParams(
            dimension_semantics=("arbitrary",),
        ),
    )(ts, osc, td, od)
    return out[0, 0]


# speedup vs baseline: 1.3998x; 1.0066x over previous
import jax
import jax.numpy as jnp
from jax.experimental import pallas as pl
from jax.experimental.pallas import tpu as pltpu

_N = 262144
_EPS = 1e-7
_ROWS = _N // 128
_DROWS = 4 * _ROWS
_BLK = 1024
_DBLK = 4 * _BLK
_STEPS = _ROWS // _BLK
_C = 32
_NCH = _BLK // _C


def _rpn_loss_kernel(ts_ref, os_ref, td_ref, od_ref, out_ref,
                     bce_ref, val_ref, reg_ref, pos_ref):
    i = pl.program_id(0)

    bce_acc = jnp.zeros((_C, 128), jnp.float32)
    val_acc = jnp.zeros((_C, 128), jnp.float32)
    reg_acc = jnp.zeros((_C, 128), jnp.float32)
    pos_acc = jnp.zeros((_C, 128), jnp.float32)

    for k in range(_NCH):
        ts = ts_ref[k * _C:(k + 1) * _C, :]
        osc = os_ref[k * _C:(k + 1) * _C, :]
        valid = (ts != -1.0).astype(jnp.float32)
        pos = ts > 0.0
        p_star = pos.astype(jnp.float32)
        o = jnp.clip(osc, _EPS, 1.0 - _EPS)
        bce = -jnp.log(jnp.where(pos, o, 1.0 - o))
        bce_acc += bce * valid
        val_acc += valid
        pos_acc += p_star

        for j in range(4):
            r0 = 4 * k * _C + j * _C
            pj = p_star[j * (_C // 4):(j + 1) * (_C // 4), :]
            mask = jnp.broadcast_to(
                pj[:, None, :], (_C // 4, 4, 128)).reshape(_C, 128)
            d = jnp.abs(od_ref[r0:r0 + _C, :] - td_ref[r0:r0 + _C, :])
            m = jnp.minimum(d, 1.0)
            reg_acc += (m * (d - 0.5 * m)) * mask

    @pl.when(i == 0)
    def _init():
        bce_ref[...] = bce_acc
        val_ref[...] = val_acc
        reg_ref[...] = reg_acc
        pos_ref[...] = pos_acc

    @pl.when(i > 0)
    def _accum():
        bce_ref[...] += bce_acc
        val_ref[...] += val_acc
        reg_ref[...] += reg_acc
        pos_ref[...] += pos_acc

    @pl.when(i == _STEPS - 1)
    def _finalize():
        cls_loss = jnp.sum(bce_ref[...]) / jnp.maximum(jnp.sum(val_ref[...]), 1.0)
        reg_loss = 10.0 * jnp.sum(reg_ref[...]) / jnp.maximum(_EPS, jnp.sum(pos_ref[...]))
        out_ref[0, 0] = cls_loss + reg_loss


def kernel(target_deltas, target_scores, output_deltas, output_scores):
    ts = target_scores.reshape(_ROWS, 128)
    osc = output_scores.reshape(_ROWS, 128)
    td = jnp.transpose(target_deltas.reshape(_ROWS, 128, 4), (0, 2, 1)).reshape(_DROWS, 128)
    od = jnp.transpose(output_deltas.reshape(_ROWS, 128, 4), (0, 2, 1)).reshape(_DROWS, 128)

    out = pl.pallas_call(
        _rpn_loss_kernel,
        grid=(_STEPS,),
        in_specs=[
            pl.BlockSpec((_BLK, 128), lambda i: (i, 0)),
            pl.BlockSpec((_BLK, 128), lambda i: (i, 0)),
            pl.BlockSpec((_DBLK, 128), lambda i: (i, 0)),
            pl.BlockSpec((_DBLK, 128), lambda i: (i, 0)),
        ],
        out_specs=pl.BlockSpec((1, 1), lambda i: (0, 0), memory_space=pltpu.SMEM),
        out_shape=jax.ShapeDtypeStruct((1, 1), jnp.float32),
        scratch_shapes=[
            pltpu.VMEM((_C, 128), jnp.float32),
            pltpu.VMEM((_C, 128), jnp.float32),
            pltpu.VMEM((_C, 128), jnp.float32),
            pltpu.VMEM((_C, 128), jnp.float32),
        ],
        compiler_params=pltpu.CompilerParams(
            dimension_semantics=("arbitrary",),
        ),
    )(ts, osc, td, od)
    return out[0, 0]
